# baseline (speedup 1.0000x reference)
import functools

import jax
import jax.numpy as jnp
from jax import lax
from jax.experimental import pallas as pl
from jax.experimental.pallas import tpu as pltpu
from jax.experimental.pallas import tpu_sc as plsc

GAMMA = 12.0
HIDDEN = 64
LANES = 16
NC = 2
NS = 16
NW = NC * NS


def _sc_body(heads, rels, tails, etab, rtab, wb, out,
             hidx, ridx, tidx, hrow, rrow, trow, wbv, outv,
             sem_h, sem_r, sem_t, b_per_w):
    wid = lax.axis_index("s") * NC + lax.axis_index("c")
    base = wid * b_per_w

    pltpu.sync_copy(heads.at[pl.ds(base, b_per_w)], hidx)
    pltpu.sync_copy(rels.at[pl.ds(base, b_per_w)], ridx)
    pltpu.sync_copy(tails.at[pl.ds(base, b_per_w)], tidx)
    pltpu.sync_copy(wb, wbv)

    cp_h = pltpu.async_copy(etab.at[hidx], hrow, sem_h)
    cp_r = pltpu.async_copy(rtab.at[ridx], rrow, sem_r)
    cp_t = pltpu.async_copy(etab.at[tidx], trow, sem_t)
    cp_h.wait()
    cp_r.wait()
    cp_t.wait()

    riota = lax.iota(jnp.int32, LANES)
    bvec = wbv[HIDDEN]

    def group_body(g, carry):
        rows = riota + g * LANES
        acc_abs = jnp.zeros((LANES,), jnp.float32)
        acc_dot = jnp.zeros((LANES,), jnp.float32)
        for d in range(HIDDEN):
            cold = jnp.full((LANES,), d, jnp.int32)
            hv = plsc.load_gather(hrow, [rows, cold])
            rv = plsc.load_gather(rrow, [rows, cold])
            tv = plsc.load_gather(trow, [rows, cold])
            z = (hv + rv) - tv
            acc_abs = acc_abs + jnp.abs(z)
            acc_dot = acc_dot + z * wbv[d]
        s = acc_dot + bvec
        dcoef = 1.0 / (1.0 + jnp.exp(-s))
        score = GAMMA - dcoef * acc_abs
        outv[pl.ds(g * LANES, LANES)] = score
        return carry

    lax.fori_loop(0, b_per_w // LANES, group_body, 0)

    pltpu.sync_copy(outv, out.at[pl.ds(base, b_per_w)])


@functools.partial(jax.jit, static_argnames=())
def _run(heads, rels, tails, etab, rtab, wb):
    batch = heads.shape[0]
    b_per_w = batch // NW
    mesh = plsc.VectorSubcoreMesh(core_axis_name="c", subcore_axis_name="s")
    kern = functools.partial(
        pl.kernel,
        out_type=jax.ShapeDtypeStruct((batch,), jnp.float32),
        mesh=mesh,
        compiler_params=pltpu.CompilerParams(
            needs_layout_passes=False, use_tc_tiling_on_sc=False),
        scratch_types=[
            pltpu.VMEM((b_per_w,), jnp.int32),
            pltpu.VMEM((b_per_w,), jnp.int32),
            pltpu.VMEM((b_per_w,), jnp.int32),
            pltpu.VMEM((b_per_w, HIDDEN), jnp.float32),
            pltpu.VMEM((b_per_w, HIDDEN), jnp.float32),
            pltpu.VMEM((b_per_w, HIDDEN), jnp.float32),
            pltpu.VMEM((HIDDEN + 1, LANES), jnp.float32),
            pltpu.VMEM((b_per_w,), jnp.float32),
            pltpu.SemaphoreType.DMA,
            pltpu.SemaphoreType.DMA,
            pltpu.SemaphoreType.DMA,
        ],
    )(functools.partial(_sc_body, b_per_w=b_per_w))
    return kern(heads, rels, tails, etab, rtab, wb)


def kernel(sample, entity_embedding, relation_embedding, D_w, D_b):
    heads = sample[:, 0]
    rels = sample[:, 1]
    tails = sample[:, 2]
    wb = jnp.broadcast_to(
        jnp.concatenate([D_w[:, 0], D_b])[:, None], (HIDDEN + 1, LANES))
    out = _run(heads, rels, tails, entity_embedding, relation_embedding, wb)
    return out[:, None]
